# Initial kernel scaffold; baseline (speedup 1.0000x reference)
import jax
import jax.numpy as jnp
from jax.experimental import pallas as pl


def kernel(x, coords, Wq, Wk, Wv, ln_g, ln_b, Wrpe, brpe, Wo):
    raise NotImplementedError("write your pallas kernel here")


# speedup vs baseline: 1.4332x; 1.4332x over previous
import numpy as np
import jax
import jax.numpy as jnp
from jax import lax
from jax.experimental import pallas as pl
from jax.experimental.pallas import tpu as pltpu

H = 8
HD = 64
BLOCK = 128
N_HASHES = 2
NUM_W = 8
CDIM = 6
N = 8192
NB = N // BLOCK
PEPAD = 8
SCALE = 1.0 / np.sqrt(HD + CDIM)


def _attn_block(xs_ref, pes_ref, pest_ref, wq_ref, wk_ref, wv_ref, wo_ref,
                g_ref, b_ref, weff_ref, out_ref):
    xb = xs_ref[0, 0]
    pe = pes_ref[0, 0]
    pet = pest_ref[0, 0]

    mu = jnp.mean(xb, axis=-1, keepdims=True)
    xc = xb - mu
    var = jnp.mean(xc * xc, axis=-1, keepdims=True)
    xn = xc * lax.rsqrt(var + 1e-5) * g_ref[0:1, :] + b_ref[0:1, :]

    dot = lambda a, b: lax.dot_general(
        a, b, (((1,), (0,)), ((), ())), preferred_element_type=jnp.float32)
    dott = lambda a, b: lax.dot_general(
        a, b, (((1,), (1,)), ((), ())), preferred_element_type=jnp.float32)

    q = dot(xn, wq_ref[...])
    k = dot(xn, wk_ref[...])
    v = dot(xn, wv_ref[...])
    pp = dott(pe, pe)

    ds = []
    for f in range(CDIM - 1):
        colv = jnp.broadcast_to(pe[:, f:f + 1], (BLOCK, BLOCK))
        rowv = jnp.broadcast_to(pet[f:f + 1, :], (BLOCK, BLOCK))
        ds.append(jnp.abs(colv - rowv))

    outs = []
    for h in range(H):
        bias = jnp.full((BLOCK, BLOCK), weff_ref[CDIM - 1, h], jnp.float32)
        for f in range(CDIM - 1):
            bias = bias + ds[f] * weff_ref[f, h]
        qh = q[:, h * HD:(h + 1) * HD]
        kh = k[:, h * HD:(h + 1) * HD]
        vh = v[:, h * HD:(h + 1) * HD]
        s = (dott(qh, kh) + pp) * SCALE + bias
        m = jnp.max(s, axis=-1, keepdims=True)
        e = jnp.exp(s - m)
        w = e / jnp.sum(e, axis=-1, keepdims=True)
        outs.append(dot(w, vh))
    ob = jnp.concatenate(outs, axis=-1)
    out_ref[0, 0] = dot(ob, wo_ref[...]) * (1.0 / N_HASHES)


def _run_attention(xs, pes, pest, Wq, Wk, Wv, Wo, g2, b2, weff):
    blk = lambda *shape: shape
    wspec = lambda s: pl.BlockSpec(s, lambda r, b: (0,) * len(s))
    return pl.pallas_call(
        _attn_block,
        grid=(N_HASHES, NB),
        in_specs=[
            pl.BlockSpec((1, 1, BLOCK, HD), lambda r, b: (r, b, 0, 0)),
            pl.BlockSpec((1, 1, BLOCK, PEPAD), lambda r, b: (r, b, 0, 0)),
            pl.BlockSpec((1, 1, PEPAD, BLOCK), lambda r, b: (r, b, 0, 0)),
            wspec(blk(HD, H * HD)),
            wspec(blk(HD, H * HD)),
            wspec(blk(HD, H * HD)),
            wspec(blk(H * HD, HD)),
            wspec(blk(1, HD)),
            wspec(blk(1, HD)),
            pl.BlockSpec(memory_space=pltpu.SMEM),
        ],
        out_specs=pl.BlockSpec((1, 1, BLOCK, HD), lambda r, b: (r, b, 0, 0)),
        out_shape=jax.ShapeDtypeStruct((N_HASHES, NB, BLOCK, HD), jnp.float32),
        compiler_params=pltpu.CompilerParams(
            dimension_semantics=("parallel", "parallel")),
    )(xs, pes, pest, Wq, Wk, Wv, Wo, g2, b2, weff)


def kernel(x, coords, Wq, Wk, Wv, ln_g, ln_b, Wrpe, brpe, Wo):
    R = jax.random.normal(jax.random.key(42), (CDIM, N_HASHES), dtype=jnp.float32)
    proj = coords @ R
    orders = jnp.argsort(proj, axis=0).T

    pe_pad = jnp.pad(coords, ((0, 0), (0, PEPAD - CDIM)))
    xs = x[orders.reshape(-1)].reshape(N_HASHES, NB, BLOCK, HD)
    pes = pe_pad[orders.reshape(-1)].reshape(N_HASHES, NB, BLOCK, PEPAD)
    pest = pes.transpose(0, 1, 3, 2)

    W_eff = Wrpe.reshape(CDIM - 1, NUM_W, H, HD).sum(axis=(1, 3))
    b_eff = brpe.reshape(H, HD).sum(-1)
    weff = jnp.zeros((PEPAD, H), jnp.float32)
    weff = weff.at[:CDIM - 1].set(W_eff).at[CDIM - 1].set(b_eff)

    ys = _run_attention(xs, pes, pest, Wq, Wk, Wv, Wo,
                        ln_g.reshape(1, HD), ln_b.reshape(1, HD), weff)
    ys = ys.reshape(N_HASHES, N, HD)

    zero = jnp.zeros((N, HD), jnp.float32)
    y0 = zero.at[orders[0]].set(ys[0])
    y1 = zero.at[orders[1]].set(ys[1])
    return y0 + y1


# speedup vs baseline: 1.5391x; 1.0739x over previous
import functools
import numpy as np
import jax
import jax.numpy as jnp
from jax import lax
from jax.experimental import pallas as pl
from jax.experimental.pallas import tpu as pltpu
from jax.experimental.pallas import tpu_sc as plsc

H = 8
HD = 64
BLOCK = 128
N_HASHES = 2
NUM_W = 8
CDIM = 6
N = 8192
NB = N // BLOCK
D = 128
PEPAD = 8
SCALE = 1.0 / np.sqrt(HD + CDIM)

_SC = plsc.get_sparse_core_info()
NW = _SC.num_cores * _SC.num_subcores
_MESH = plsc.VectorSubcoreMesh(core_axis_name="c", subcore_axis_name="s")


_GB = (N_HASHES * N) // NW
_GC = _GB // BLOCK


@functools.partial(
    pl.kernel, mesh=_MESH,
    out_type=jax.ShapeDtypeStruct((N_HASHES * N, D), jnp.float32),
    scratch_types=[
        pltpu.VMEM((_GC, BLOCK), jnp.int32),
        pltpu.VMEM((_GB, D), jnp.float32),
        pltpu.SemaphoreType.DMA,
    ],
)
def _sc_gather(table_hbm, idx_hbm, out_hbm, idx_v, rows_v, sem):
    wid = lax.axis_index("s") * _SC.num_cores + lax.axis_index("c")
    pltpu.sync_copy(idx_hbm.at[pl.ds(wid * _GC, _GC)], idx_v)
    cps = [pltpu.async_copy(table_hbm.at[idx_v.at[j]],
                            rows_v.at[pl.ds(j * BLOCK, BLOCK)], sem)
           for j in range(_GC)]
    for c in cps:
        c.wait()
    pltpu.sync_copy(rows_v, out_hbm.at[pl.ds(wid * _GB, _GB)])


_CB = N // NW
_CC = _CB // BLOCK


@functools.partial(
    pl.kernel, mesh=_MESH,
    out_type=jax.ShapeDtypeStruct((N, D), jnp.float32),
    scratch_types=[
        pltpu.VMEM((_CC, BLOCK), jnp.int32),
        pltpu.VMEM((_CC, BLOCK), jnp.int32),
        pltpu.VMEM((_CB, D), jnp.float32),
        pltpu.VMEM((_CB, D), jnp.float32),
        pltpu.SemaphoreType.DMA,
    ],
)
def _sc_combine(ys_hbm, inv_hbm, out_hbm, i0_v, i1_v, a_v, b_v, sem):
    wid = lax.axis_index("s") * _SC.num_cores + lax.axis_index("c")
    nrow = N // BLOCK
    pltpu.sync_copy(inv_hbm.at[pl.ds(wid * _CC, _CC)], i0_v)
    pltpu.sync_copy(inv_hbm.at[pl.ds(nrow + wid * _CC, _CC)], i1_v)
    cps = []
    for j in range(_CC):
        cps.append(pltpu.async_copy(ys_hbm.at[i0_v.at[j]],
                                    a_v.at[pl.ds(j * BLOCK, BLOCK)], sem))
        cps.append(pltpu.async_copy(ys_hbm.at[i1_v.at[j]],
                                    b_v.at[pl.ds(j * BLOCK, BLOCK)], sem))
    for c in cps:
        c.wait()

    def body(i, carry):
        for j in range(D // 16):
            sl = pl.ds(j * 16, 16)
            a_v[i, sl] = a_v[i, sl] + b_v[i, sl]
        return carry

    lax.fori_loop(0, _CB, body, 0)
    pltpu.sync_copy(a_v, out_hbm.at[pl.ds(wid * _CB, _CB)])


def _attn_block(g_ref, wq_ref, wk_ref, wv_ref, wo_ref, gg_ref, bb_ref,
                weff_ref, out_ref):
    gb = g_ref[0, 0]
    xb = gb[:, :HD]
    pe = gb[:, HD:HD + PEPAD]

    mu = jnp.mean(xb, axis=-1, keepdims=True)
    xc = xb - mu
    var = jnp.mean(xc * xc, axis=-1, keepdims=True)
    xn = xc * lax.rsqrt(var + 1e-5) * gg_ref[0:1, :] + bb_ref[0:1, :]

    dot = lambda a, b: lax.dot_general(
        a, b, (((1,), (0,)), ((), ())), preferred_element_type=jnp.float32)
    dott = lambda a, b: lax.dot_general(
        a, b, (((1,), (1,)), ((), ())), preferred_element_type=jnp.float32)

    q = dot(xn, wq_ref[...])
    k = dot(xn, wk_ref[...])
    v = dot(xn, wv_ref[...])
    pp = dott(pe, pe)

    ones_col = jnp.ones((BLOCK, 1), jnp.float32)
    ds = []
    for f in range(CDIM - 1):
        colv = jnp.broadcast_to(pe[:, f:f + 1], (BLOCK, BLOCK))
        rowv = dott(ones_col, pe[:, f:f + 1])
        ds.append(jnp.abs(colv - rowv))

    outs = []
    for h in range(H):
        bias = jnp.full((BLOCK, BLOCK), weff_ref[CDIM - 1, h], jnp.float32)
        for f in range(CDIM - 1):
            bias = bias + ds[f] * weff_ref[f, h]
        qh = q[:, h * HD:(h + 1) * HD]
        kh = k[:, h * HD:(h + 1) * HD]
        vh = v[:, h * HD:(h + 1) * HD]
        s = (dott(qh, kh) + pp) * SCALE + bias
        m = jnp.max(s, axis=-1, keepdims=True)
        e = jnp.exp(s - m)
        w = e / jnp.sum(e, axis=-1, keepdims=True)
        outs.append(dot(w, vh))
    ob = jnp.concatenate(outs, axis=-1)
    out_ref[0, 0] = dot(ob, wo_ref[...]) * (1.0 / N_HASHES)


def _run_attention(g, Wq, Wk, Wv, Wo_pad, g2, b2, weff):
    wspec = lambda s: pl.BlockSpec(s, lambda r, b: (0,) * len(s))
    return pl.pallas_call(
        _attn_block,
        grid=(N_HASHES, NB),
        in_specs=[
            pl.BlockSpec((1, 1, BLOCK, D), lambda r, b: (r, b, 0, 0)),
            wspec((HD, H * HD)),
            wspec((HD, H * HD)),
            wspec((HD, H * HD)),
            wspec((H * HD, D)),
            wspec((1, HD)),
            wspec((1, HD)),
            pl.BlockSpec(memory_space=pltpu.SMEM),
        ],
        out_specs=pl.BlockSpec((1, 1, BLOCK, D), lambda r, b: (r, b, 0, 0)),
        out_shape=jax.ShapeDtypeStruct((N_HASHES, NB, BLOCK, D), jnp.float32),
        compiler_params=pltpu.CompilerParams(
            dimension_semantics=("parallel", "parallel")),
    )(g, Wq, Wk, Wv, Wo_pad, g2, b2, weff)


def kernel(x, coords, Wq, Wk, Wv, ln_g, ln_b, Wrpe, brpe, Wo):
    R = jax.random.normal(jax.random.key(42), (CDIM, N_HASHES), dtype=jnp.float32)
    proj = coords @ R
    orders = jnp.argsort(proj, axis=0).T

    iota = jnp.arange(N, dtype=jnp.int32)
    inv0 = jnp.zeros((N,), jnp.int32).at[orders[0]].set(iota)
    inv1 = jnp.zeros((N,), jnp.int32).at[orders[1]].set(iota)
    inv_cat = jnp.concatenate([inv0, inv1 + N]).reshape(-1, BLOCK)

    table = jnp.concatenate(
        [x, coords, jnp.zeros((N, D - HD - CDIM), jnp.float32)], axis=1)

    g = _sc_gather(table, orders.reshape(-1, BLOCK))
    g = g.reshape(N_HASHES, NB, BLOCK, D)

    W_eff = Wrpe.reshape(CDIM - 1, NUM_W, H, HD).sum(axis=(1, 3))
    b_eff = brpe.reshape(H, HD).sum(-1)
    weff = jnp.zeros((PEPAD, H), jnp.float32)
    weff = weff.at[:CDIM - 1].set(W_eff).at[CDIM - 1].set(b_eff)

    Wo_pad = jnp.pad(Wo, ((0, 0), (0, D - HD)))

    ys = _run_attention(g, Wq, Wk, Wv, Wo_pad,
                        ln_g.reshape(1, HD), ln_b.reshape(1, HD), weff)
    y = _sc_combine(ys.reshape(N_HASHES * N, D), inv_cat)
    return y[:, :HD]


# speedup vs baseline: 2.5730x; 1.6718x over previous
import functools
import numpy as np
import jax
import jax.numpy as jnp
from jax import lax
from jax.experimental import pallas as pl
from jax.experimental.pallas import tpu as pltpu
from jax.experimental.pallas import tpu_sc as plsc

H = 8
HD = 64
BLOCK = 128
N_HASHES = 2
NUM_W = 8
CDIM = 6
N = 8192
NB = N // BLOCK
D = 128
PEPAD = 8
SCALE = 1.0 / np.sqrt(HD + CDIM)

_SC = plsc.get_sparse_core_info()
NW = _SC.num_cores * _SC.num_subcores
_MESH = plsc.VectorSubcoreMesh(core_axis_name="c", subcore_axis_name="s")


_GB = (N_HASHES * N) // NW
_GC = _GB // BLOCK


@functools.partial(
    pl.kernel, mesh=_MESH,
    out_type=jax.ShapeDtypeStruct((N_HASHES * N, D), jnp.float32),
    scratch_types=[
        pltpu.VMEM((_GC, BLOCK), jnp.int32),
        pltpu.VMEM((_GB, D), jnp.float32),
        pltpu.SemaphoreType.DMA,
    ],
)
def _sc_gather(table_hbm, idx_hbm, out_hbm, idx_v, rows_v, sem):
    wid = lax.axis_index("s") * _SC.num_cores + lax.axis_index("c")
    pltpu.sync_copy(idx_hbm.at[pl.ds(wid * _GC, _GC)], idx_v)
    cps = [pltpu.async_copy(table_hbm.at[idx_v.at[j]],
                            rows_v.at[pl.ds(j * BLOCK, BLOCK)], sem)
           for j in range(_GC)]
    for c in cps:
        c.wait()
    pltpu.sync_copy(rows_v, out_hbm.at[pl.ds(wid * _GB, _GB)])


_CB = N // NW
_CC = _CB // BLOCK


@functools.partial(
    pl.kernel, mesh=_MESH,
    out_type=jax.ShapeDtypeStruct((N, D), jnp.float32),
    scratch_types=[
        pltpu.VMEM((_CC, BLOCK), jnp.int32),
        pltpu.VMEM((_CC, BLOCK), jnp.int32),
        pltpu.VMEM((_CB, D), jnp.float32),
        pltpu.VMEM((_CB, D), jnp.float32),
        pltpu.SemaphoreType.DMA,
    ],
)
def _sc_combine(ys_hbm, inv_hbm, out_hbm, i0_v, i1_v, a_v, b_v, sem):
    wid = lax.axis_index("s") * _SC.num_cores + lax.axis_index("c")
    nrow = N // BLOCK
    pltpu.sync_copy(inv_hbm.at[pl.ds(wid * _CC, _CC)], i0_v)
    pltpu.sync_copy(inv_hbm.at[pl.ds(nrow + wid * _CC, _CC)], i1_v)
    cps = []
    for j in range(_CC):
        cps.append(pltpu.async_copy(ys_hbm.at[i0_v.at[j]],
                                    a_v.at[pl.ds(j * BLOCK, BLOCK)], sem))
        cps.append(pltpu.async_copy(ys_hbm.at[i1_v.at[j]],
                                    b_v.at[pl.ds(j * BLOCK, BLOCK)], sem))
    for c in cps:
        c.wait()

    def body(i, carry):
        for j in range(D // 16):
            sl = pl.ds(j * 16, 16)
            a_v[i, sl] = a_v[i, sl] + b_v[i, sl]
        return carry

    lax.fori_loop(0, _CB, body, 0)
    pltpu.sync_copy(a_v, out_hbm.at[pl.ds(wid * _CB, _CB)])


MB = 4


def _attn_block(g_ref, wq_ref, wk_ref, wv_ref, wo_ref, gg_ref, bb_ref,
                weff_ref, out_ref):
    for mb in range(MB):
        _attn_one(g_ref, wq_ref, wk_ref, wv_ref, wo_ref, gg_ref, bb_ref,
                  weff_ref, out_ref, mb)


def _attn_one(g_ref, wq_ref, wk_ref, wv_ref, wo_ref, gg_ref, bb_ref,
              weff_ref, out_ref, mb):
    bf = jnp.bfloat16
    gb = g_ref[0, mb]
    xb = gb[:, :HD]
    pe = gb[:, HD:HD + PEPAD]

    mu = jnp.mean(xb, axis=-1, keepdims=True)
    xc = xb - mu
    var = jnp.mean(xc * xc, axis=-1, keepdims=True)
    xn = (xc * lax.rsqrt(var + 1e-5) * gg_ref[0:1, :] + bb_ref[0:1, :]).astype(bf)

    f32 = jnp.float32
    dot = lambda a, b: lax.dot_general(
        a, b, (((1,), (0,)), ((), ())), preferred_element_type=f32)
    dott = lambda a, b: lax.dot_general(
        a, b, (((1,), (1,)), ((), ())), preferred_element_type=f32)

    q = dot(xn, wq_ref[...]).astype(bf)
    k = dot(xn, wk_ref[...]).astype(bf)
    v = dot(xn, wv_ref[...]).astype(bf)
    peb = pe.astype(bf)
    pp = dott(peb, peb)

    pet = lax.transpose(pe, (1, 0))
    ds = []
    for f in range(CDIM - 1):
        colv = jnp.broadcast_to(pe[:, f:f + 1], (BLOCK, BLOCK))
        rowv = jnp.broadcast_to(pet[f:f + 1, :], (BLOCK, BLOCK))
        ds.append(jnp.abs(colv - rowv).astype(bf))
    dbig = jnp.concatenate(ds + [jnp.ones((BLOCK, 8), bf)], axis=-1)
    bias_all = dot(dbig, weff_ref[...])

    ss = [(dott(q[:, h * HD:(h + 1) * HD], k[:, h * HD:(h + 1) * HD]) + pp)
          * SCALE + bias_all[:, h * BLOCK:(h + 1) * BLOCK] for h in range(H)]
    ms = [jnp.max(s, axis=-1, keepdims=True) for s in ss]
    es = [jnp.exp(s - m) for s, m in zip(ss, ms)]
    rs = [jnp.reciprocal(jnp.sum(e, axis=-1, keepdims=True)) for e in es]
    ws = [(e * r).astype(bf) for e, r in zip(es, rs)]
    outs = [dot(w, v[:, h * HD:(h + 1) * HD]).astype(bf)
            for h, w in enumerate(ws)]
    ob = jnp.concatenate(outs, axis=-1)
    out_ref[0, mb] = dot(ob, wo_ref[...]) * (1.0 / N_HASHES)


def _run_attention(g, Wq, Wk, Wv, Wo_pad, g2, b2, weff):
    wspec = lambda s: pl.BlockSpec(s, lambda r, b: (0,) * len(s))
    return pl.pallas_call(
        _attn_block,
        grid=(N_HASHES, NB // MB),
        in_specs=[
            pl.BlockSpec((1, MB, BLOCK, D), lambda r, b: (r, b, 0, 0)),
            wspec((HD, H * HD)),
            wspec((HD, H * HD)),
            wspec((HD, H * HD)),
            wspec((H * HD, D)),
            wspec((1, HD)),
            wspec((1, HD)),
            wspec(((CDIM - 1) * BLOCK + 8, H * BLOCK)),
        ],
        out_specs=pl.BlockSpec((1, MB, BLOCK, D), lambda r, b: (r, b, 0, 0)),
        out_shape=jax.ShapeDtypeStruct((N_HASHES, NB, BLOCK, D), jnp.float32),
        compiler_params=pltpu.CompilerParams(
            dimension_semantics=("parallel", "parallel")),
    )(g, Wq, Wk, Wv, Wo_pad, g2, b2, weff)


def kernel(x, coords, Wq, Wk, Wv, ln_g, ln_b, Wrpe, brpe, Wo):
    R = jax.random.normal(jax.random.key(42), (CDIM, N_HASHES), dtype=jnp.float32)
    proj = coords @ R
    orders = jnp.argsort(proj, axis=0).T

    iota = jnp.arange(N, dtype=jnp.int32)
    inv0 = jnp.zeros((N,), jnp.int32).at[orders[0]].set(iota)
    inv1 = jnp.zeros((N,), jnp.int32).at[orders[1]].set(iota)
    inv_cat = jnp.concatenate([inv0, inv1 + N]).reshape(-1, BLOCK)

    table = jnp.concatenate(
        [x, coords, jnp.zeros((N, D - HD - CDIM), jnp.float32)], axis=1)

    g = _sc_gather(table, orders.reshape(-1, BLOCK))
    g = g.reshape(N_HASHES, NB, BLOCK, D)

    W_eff = Wrpe.reshape(CDIM - 1, NUM_W, H, HD).sum(axis=(1, 3))
    b_eff = brpe.reshape(H, HD).sum(-1)
    eye = jnp.eye(BLOCK, dtype=jnp.float32)
    wb = jnp.einsum('fh,jJ->fjhJ', W_eff, eye).reshape(
        (CDIM - 1) * BLOCK, H * BLOCK)
    weff = jnp.concatenate(
        [wb, jnp.repeat(b_eff, BLOCK)[None, :],
         jnp.zeros((7, H * BLOCK), jnp.float32)], axis=0).astype(jnp.bfloat16)

    Wo_pad = jnp.pad(Wo, ((0, 0), (0, D - HD))).astype(jnp.bfloat16)

    ys = _run_attention(g, Wq.astype(jnp.bfloat16), Wk.astype(jnp.bfloat16),
                        Wv.astype(jnp.bfloat16), Wo_pad,
                        ln_g.reshape(1, HD), ln_b.reshape(1, HD), weff)
    y = _sc_combine(ys.reshape(N_HASHES * N, D), inv_cat)
    return y[:, :HD]


# speedup vs baseline: 2.6628x; 1.0349x over previous
import functools
import numpy as np
import jax
import jax.numpy as jnp
from jax import lax
from jax.experimental import pallas as pl
from jax.experimental.pallas import tpu as pltpu
from jax.experimental.pallas import tpu_sc as plsc

H = 8
HD = 64
BLOCK = 128
N_HASHES = 2
NUM_W = 8
CDIM = 6
N = 8192
NB = N // BLOCK
D = 128
PEPAD = 8
SCALE = 1.0 / np.sqrt(HD + CDIM)

_SC = plsc.get_sparse_core_info()
NW = _SC.num_cores * _SC.num_subcores
_MESH = plsc.VectorSubcoreMesh(core_axis_name="c", subcore_axis_name="s")


_GB = (N_HASHES * N) // NW
_GC = _GB // BLOCK


@functools.partial(
    pl.kernel, mesh=_MESH,
    out_type=jax.ShapeDtypeStruct((N_HASHES * N, D), jnp.float32),
    scratch_types=[
        pltpu.VMEM((_GC, BLOCK), jnp.int32),
        pltpu.VMEM((_GB, D), jnp.float32),
        pltpu.SemaphoreType.DMA,
    ],
)
def _sc_gather(table_hbm, idx_hbm, out_hbm, idx_v, rows_v, sem):
    wid = lax.axis_index("s") * _SC.num_cores + lax.axis_index("c")
    pltpu.sync_copy(idx_hbm.at[pl.ds(wid * _GC, _GC)], idx_v)
    cps = [pltpu.async_copy(table_hbm.at[idx_v.at[j]],
                            rows_v.at[pl.ds(j * BLOCK, BLOCK)], sem)
           for j in range(_GC)]
    for c in cps:
        c.wait()
    pltpu.sync_copy(rows_v, out_hbm.at[pl.ds(wid * _GB, _GB)])


_CB = N // NW
_CC = _CB // BLOCK


@functools.partial(
    pl.kernel, mesh=_MESH,
    out_type=jax.ShapeDtypeStruct((N, D), jnp.float32),
    scratch_types=[
        pltpu.VMEM((_CC, BLOCK), jnp.int32),
        pltpu.VMEM((_CC, BLOCK), jnp.int32),
        pltpu.VMEM((_CB, D), jnp.float32),
        pltpu.VMEM((_CB, D), jnp.float32),
        pltpu.SemaphoreType.DMA,
    ],
)
def _sc_combine(ys_hbm, inv_hbm, out_hbm, i0_v, i1_v, a_v, b_v, sem):
    wid = lax.axis_index("s") * _SC.num_cores + lax.axis_index("c")
    nrow = N // BLOCK
    pltpu.sync_copy(inv_hbm.at[pl.ds(wid * _CC, _CC)], i0_v)
    pltpu.sync_copy(inv_hbm.at[pl.ds(nrow + wid * _CC, _CC)], i1_v)
    cps = []
    for j in range(_CC):
        cps.append(pltpu.async_copy(ys_hbm.at[i0_v.at[j]],
                                    a_v.at[pl.ds(j * BLOCK, BLOCK)], sem))
        cps.append(pltpu.async_copy(ys_hbm.at[i1_v.at[j]],
                                    b_v.at[pl.ds(j * BLOCK, BLOCK)], sem))
    for c in cps:
        c.wait()

    def body(i, carry):
        for j in range(D // 16):
            sl = pl.ds(j * 16, 16)
            a_v[i, sl] = a_v[i, sl] + b_v[i, sl]
        return carry

    lax.fori_loop(0, _CB, body, 0)
    pltpu.sync_copy(a_v, out_hbm.at[pl.ds(wid * _CB, _CB)])


MB = 1


def _attn_block(g_ref, wq_ref, wk_ref, wv_ref, wo_ref, gg_ref, bb_ref,
                weff_ref, out_ref):
    for mb in range(MB):
        _attn_one(g_ref, wq_ref, wk_ref, wv_ref, wo_ref, gg_ref, bb_ref,
                  weff_ref, out_ref, mb)


def _attn_one(g_ref, wq_ref, wk_ref, wv_ref, wo_ref, gg_ref, bb_ref,
              weff_ref, out_ref, mb):
    bf = jnp.bfloat16
    gb = g_ref[0, mb]
    xb = gb[:, :HD]
    pe = gb[:, HD:HD + PEPAD]

    mu = jnp.mean(xb, axis=-1, keepdims=True)
    xc = xb - mu
    var = jnp.mean(xc * xc, axis=-1, keepdims=True)
    xn = (xc * lax.rsqrt(var + 1e-5) * gg_ref[0:1, :] + bb_ref[0:1, :]).astype(bf)

    f32 = jnp.float32
    dot = lambda a, b: lax.dot_general(
        a, b, (((1,), (0,)), ((), ())), preferred_element_type=f32)
    dott = lambda a, b: lax.dot_general(
        a, b, (((1,), (1,)), ((), ())), preferred_element_type=f32)

    q = dot(xn, wq_ref[...]).astype(bf)
    k = dot(xn, wk_ref[...]).astype(bf)
    v = dot(xn, wv_ref[...]).astype(bf)
    peb = pe.astype(bf)
    pp = dott(peb, peb)

    pet = lax.transpose(pe, (1, 0))
    ds = []
    for f in range(CDIM - 1):
        colv = jnp.broadcast_to(pe[:, f:f + 1], (BLOCK, BLOCK))
        rowv = jnp.broadcast_to(pet[f:f + 1, :], (BLOCK, BLOCK))
        ds.append(jnp.abs(colv - rowv).astype(bf))
    dbig = jnp.concatenate(ds + [jnp.ones((BLOCK, 8), bf)], axis=-1)
    bias_all = dot(dbig, weff_ref[...])

    ss = [(dott(q[:, h * HD:(h + 1) * HD], k[:, h * HD:(h + 1) * HD]) + pp)
          * SCALE + bias_all[:, h * BLOCK:(h + 1) * BLOCK] for h in range(H)]
    ms = [jnp.max(s, axis=-1, keepdims=True) for s in ss]
    es = [jnp.exp(s - m) for s, m in zip(ss, ms)]
    rs = [jnp.reciprocal(jnp.sum(e, axis=-1, keepdims=True)) for e in es]
    ws = [(e * r).astype(bf) for e, r in zip(es, rs)]
    outs = [dot(w, v[:, h * HD:(h + 1) * HD]).astype(bf)
            for h, w in enumerate(ws)]
    ob = jnp.concatenate(outs, axis=-1)
    out_ref[0, mb] = dot(ob, wo_ref[...]) * (1.0 / N_HASHES)


def _run_attention(g, Wq, Wk, Wv, Wo_pad, g2, b2, weff):
    wspec = lambda s: pl.BlockSpec(s, lambda r, b: (0,) * len(s))
    return pl.pallas_call(
        _attn_block,
        grid=(N_HASHES, NB // MB),
        in_specs=[
            pl.BlockSpec((1, MB, BLOCK, D), lambda r, b: (r, b, 0, 0)),
            wspec((HD, H * HD)),
            wspec((HD, H * HD)),
            wspec((HD, H * HD)),
            wspec((H * HD, D)),
            wspec((1, HD)),
            wspec((1, HD)),
            wspec(((CDIM - 1) * BLOCK + 8, H * BLOCK)),
        ],
        out_specs=pl.BlockSpec((1, MB, BLOCK, D), lambda r, b: (r, b, 0, 0)),
        out_shape=jax.ShapeDtypeStruct((N_HASHES, NB, BLOCK, D), jnp.float32),
        compiler_params=pltpu.CompilerParams(
            dimension_semantics=("parallel", "parallel")),
    )(g, Wq, Wk, Wv, Wo_pad, g2, b2, weff)


def kernel(x, coords, Wq, Wk, Wv, ln_g, ln_b, Wrpe, brpe, Wo):
    R = jax.random.normal(jax.random.key(42), (CDIM, N_HASHES), dtype=jnp.float32)
    proj = coords @ R
    orders = jnp.argsort(proj, axis=0).T

    iota = jnp.arange(N, dtype=jnp.int32)
    inv0 = jnp.zeros((N,), jnp.int32).at[orders[0]].set(iota)
    inv1 = jnp.zeros((N,), jnp.int32).at[orders[1]].set(iota)
    inv_cat = jnp.concatenate([inv0, inv1 + N]).reshape(-1, BLOCK)

    table = jnp.concatenate(
        [x, coords, jnp.zeros((N, D - HD - CDIM), jnp.float32)], axis=1)

    g = _sc_gather(table, orders.reshape(-1, BLOCK))
    g = g.reshape(N_HASHES, NB, BLOCK, D)

    W_eff = Wrpe.reshape(CDIM - 1, NUM_W, H, HD).sum(axis=(1, 3))
    b_eff = brpe.reshape(H, HD).sum(-1)
    eye = jnp.eye(BLOCK, dtype=jnp.float32)
    wb = jnp.einsum('fh,jJ->fjhJ', W_eff, eye).reshape(
        (CDIM - 1) * BLOCK, H * BLOCK)
    weff = jnp.concatenate(
        [wb, jnp.repeat(b_eff, BLOCK)[None, :],
         jnp.zeros((7, H * BLOCK), jnp.float32)], axis=0).astype(jnp.bfloat16)

    Wo_pad = jnp.pad(Wo, ((0, 0), (0, D - HD))).astype(jnp.bfloat16)

    ys = _run_attention(g, Wq.astype(jnp.bfloat16), Wk.astype(jnp.bfloat16),
                        Wv.astype(jnp.bfloat16), Wo_pad,
                        ln_g.reshape(1, HD), ln_b.reshape(1, HD), weff)
    y = _sc_combine(ys.reshape(N_HASHES * N, D), inv_cat)
    return y[:, :HD]
